# Initial kernel scaffold; baseline (speedup 1.0000x reference)
import jax
import jax.numpy as jnp
from jax.experimental import pallas as pl


def kernel(input_ids, embed_tokens):
    raise NotImplementedError("write your pallas kernel here")


# speedup vs baseline: 1.6286x; 1.6286x over previous
import functools

import jax
import jax.numpy as jnp
from jax import lax
from jax.experimental import pallas as pl
from jax.experimental.pallas import tpu as pltpu
from jax.experimental.pallas import tpu_sc as plsc

HID = 7168
NC = 2
NS = 16
NW = NC * NS
CHUNK = 8


def _emb_body(nchunk, idx_hbm, table_hbm, out_hbm,
              idx_v, buf0, buf1, si0, si1, so0, so1):
    wid = lax.axis_index("s") * NC + lax.axis_index("c")
    rows_per_w = nchunk * CHUNK
    base = wid * rows_per_w
    pltpu.sync_copy(idx_hbm.at[wid], idx_v)

    bufs = (buf0, buf1)
    in_sems = (si0, si1)
    out_sems = (so0, so1)
    in_copies = [None, None]
    out_copies = [None, None]

    in_copies[0] = pltpu.async_copy(table_hbm.at[idx_v.at[0]], bufs[0],
                                    in_sems[0])
    for i in range(nchunk):
        b = i % 2
        nb = (i + 1) % 2
        if i + 1 < nchunk:
            if out_copies[nb] is not None:
                out_copies[nb].wait()
            in_copies[nb] = pltpu.async_copy(table_hbm.at[idx_v.at[i + 1]],
                                             bufs[nb], in_sems[nb])
        in_copies[b].wait()
        out_copies[b] = pltpu.async_copy(
            bufs[b], out_hbm.at[pl.ds(base + i * CHUNK, CHUNK)], out_sems[b])
    for c in out_copies:
        if c is not None:
            c.wait()


@functools.partial(jax.jit, static_argnames=("ntok",))
def _emb_call(flat_idx, table, ntok):
    nchunk = ntok // (NW * CHUNK)
    mesh = plsc.VectorSubcoreMesh(core_axis_name="c", subcore_axis_name="s")
    k = functools.partial(
        pl.kernel,
        mesh=mesh,
        out_type=jax.ShapeDtypeStruct((ntok, HID), jnp.float32),
        scratch_types=[
            pltpu.VMEM((nchunk, CHUNK), jnp.int32),
            pltpu.VMEM((CHUNK, HID), jnp.float32),
            pltpu.VMEM((CHUNK, HID), jnp.float32),
            pltpu.SemaphoreType.DMA,
            pltpu.SemaphoreType.DMA,
            pltpu.SemaphoreType.DMA,
            pltpu.SemaphoreType.DMA,
        ],
    )(functools.partial(_emb_body, nchunk))
    return k(flat_idx.reshape(NW, nchunk, CHUNK), table)


def kernel(input_ids, embed_tokens):
    ntok = input_ids.size
    flat = input_ids.reshape(-1)
    out = _emb_call(flat, embed_tokens, ntok)
    return out.reshape(1, 1, ntok, embed_tokens.shape[1])
